# baseline (speedup 1.0000x reference)
import jax
import jax.numpy as jnp
from jax.experimental import pallas as pl
from jax.experimental.pallas import tpu as pltpu

_BM = 400
_DEPTH = 3


def _main_kernel(nb, a_hbm, x_hbm, wt_ref, b_ref, o_ref,
                 a_buf, x_buf, y_ref, a_sems, x_sem):
    i = pl.program_id(0)
    bm = a_buf.shape[1]

    @pl.when(i == 0)
    def _():
        for j in range(_DEPTH - 1):
            pltpu.make_async_copy(
                a_hbm.at[pl.ds(j * bm, bm), :], a_buf.at[j], a_sems.at[j]
            ).start()
        xcopy = pltpu.make_async_copy(x_hbm, x_buf, x_sem)
        xcopy.start()
        xcopy.wait()
        y_ref[...] = jnp.dot(x_buf[...], wt_ref[...],
                             preferred_element_type=jnp.float32
                             ).astype(jnp.bfloat16)

    slot = jax.lax.rem(i, _DEPTH)
    pltpu.make_async_copy(
        a_hbm.at[pl.ds(i * bm, bm), :], a_buf.at[slot], a_sems.at[slot]
    ).wait()
    acc = jnp.dot(a_buf[slot].astype(jnp.bfloat16), y_ref[...],
                  preferred_element_type=jnp.float32)
    o_ref[...] = jnp.maximum(acc + b_ref[...], 0.0)

    nxt = i + _DEPTH - 1

    @pl.when(nxt < nb)
    def _():
        nslot = jax.lax.rem(nxt, _DEPTH)
        pltpu.make_async_copy(
            a_hbm.at[pl.ds(nxt * bm, bm), :], a_buf.at[nslot],
            a_sems.at[nslot]
        ).start()


def kernel(node_features, adjacency_matrix, W, b):
    n, d_in = node_features.shape
    d_out = W.shape[0]
    wt = W.T
    b2d = b.reshape(1, d_out)
    nb = n // _BM

    import functools
    return pl.pallas_call(
        functools.partial(_main_kernel, nb),
        grid=(nb,),
        in_specs=[
            pl.BlockSpec(memory_space=pl.ANY),
            pl.BlockSpec(memory_space=pl.ANY),
            pl.BlockSpec((d_in, d_out), lambda i: (0, 0)),
            pl.BlockSpec((1, d_out), lambda i: (0, 0)),
        ],
        out_specs=pl.BlockSpec((_BM, d_out), lambda i: (i, 0)),
        out_shape=jax.ShapeDtypeStruct((n, d_out), jnp.float32),
        scratch_shapes=[
            pltpu.VMEM((_DEPTH, _BM, n), jnp.float32),
            pltpu.VMEM((n, d_in), jnp.float32),
            pltpu.VMEM((n, d_out), jnp.bfloat16),
            pltpu.SemaphoreType.DMA((_DEPTH,)),
            pltpu.SemaphoreType.DMA,
        ],
    )(adjacency_matrix, node_features, wt, b2d)


# speedup vs baseline: 1.0388x; 1.0388x over previous
import jax
import jax.numpy as jnp
from jax.experimental import pallas as pl
from jax.experimental.pallas import tpu as pltpu


def _main_kernel(a_ref, x_ref, wt_ref, b_ref, o_ref, y_ref):
    @pl.when(pl.program_id(0) == 0)
    def _():
        y_ref[...] = jnp.dot(x_ref[...], wt_ref[...],
                             preferred_element_type=jnp.float32
                             ).astype(jnp.bfloat16)

    acc = jnp.dot(a_ref[...].astype(jnp.bfloat16), y_ref[...],
                  preferred_element_type=jnp.float32)
    o_ref[...] = jnp.maximum(acc + b_ref[...], 0.0)


def kernel(node_features, adjacency_matrix, W, b):
    n, d_in = node_features.shape
    d_out = W.shape[0]
    wt = W.T
    b2d = b.reshape(1, d_out)

    bm = 400
    return pl.pallas_call(
        _main_kernel,
        grid=(n // bm,),
        in_specs=[
            pl.BlockSpec((bm, n), lambda i: (i, 0)),
            pl.BlockSpec((n, d_in), lambda i: (0, 0)),
            pl.BlockSpec((d_in, d_out), lambda i: (0, 0)),
            pl.BlockSpec((1, d_out), lambda i: (0, 0)),
        ],
        out_specs=pl.BlockSpec((bm, d_out), lambda i: (i, 0)),
        out_shape=jax.ShapeDtypeStruct((n, d_out), jnp.float32),
        scratch_shapes=[pltpu.VMEM((n, d_out), jnp.bfloat16)],
    )(adjacency_matrix, node_features, wt, b2d)
